# baseline (speedup 1.0000x reference)
import functools

import jax
import jax.numpy as jnp
from jax import lax
from jax.experimental import pallas as pl
from jax.experimental.pallas import tpu as pltpu
from jax.experimental.pallas import tpu_sc as plsc

B = 8
A = 262144
N = B * A
NC = 2
NS = 16
NW = NC * NS
PER_W = N // NW
CH = 8192
NCHUNK = PER_W // CH
L = 16

_LOG1P = (
    3.5075520536942406e-06,
    0.999792435728606,
    -0.49697791116761014,
    0.31459053537083104,
    -0.18878267362071732,
    0.08172680837495,
    -0.017208061121084715,
)


def _ce_body(match_hbm, logits_hbm, out_hbm, m_buf, l_buf, stage, sem):
    wid = lax.axis_index("s") * NC + lax.axis_index("c")
    base = wid * PER_W
    ev = lax.iota(jnp.int32, L) * 2

    sum_vec = jnp.zeros((L,), jnp.float32)
    cnt_vec = jnp.zeros((L,), jnp.float32)

    for c in range(NCHUNK):
        off = base + c * CH
        pltpu.sync_copy(match_hbm.at[pl.ds(off, CH)], m_buf)
        pltpu.sync_copy(logits_hbm.at[pl.ds(2 * off, 2 * CH)], l_buf)

        def body(j, carry):
            s, n = carry
            m = m_buf[pl.ds(j * L, L)]
            idx = ev + j * (2 * L)
            l0 = plsc.load_gather(l_buf, [idx])
            l1 = plsc.load_gather(l_buf, [idx + 1])
            d = l0 - l1
            ax = jnp.abs(d)
            t = jnp.exp(-ax)
            p = _LOG1P[6]
            for k in (5, 4, 3, 2, 1, 0):
                p = p * t + _LOG1P[k]
            c1 = m == 1
            r = jnp.where(c1 == (d > 0), ax, 0.0)
            ce = p + r
            valid = m != 0
            s = s + jnp.where(valid, ce, 0.0)
            n = n + jnp.where(valid, 1.0, 0.0)
            return s, n

        sum_vec, cnt_vec = lax.fori_loop(0, CH // L, body, (sum_vec, cnt_vec))

    stage[pl.ds(0, L)] = sum_vec
    stage[pl.ds(L, L)] = cnt_vec
    pltpu.sync_copy(stage, out_hbm.at[wid])


@jax.jit
def kernel(rpn_match, rpn_class_logits):
    match_flat = rpn_match.reshape(N)
    logits_flat = rpn_class_logits.reshape(2 * N)

    mesh = plsc.VectorSubcoreMesh(core_axis_name="c", subcore_axis_name="s")
    partials = pl.kernel(
        _ce_body,
        out_type=jax.ShapeDtypeStruct((NW, 2 * L), jnp.float32),
        mesh=mesh,
        compiler_params=pltpu.CompilerParams(needs_layout_passes=False),
        scratch_types=[
            pltpu.VMEM((CH,), jnp.int32),
            pltpu.VMEM((2 * CH,), jnp.float32),
            pltpu.VMEM((2 * L,), jnp.float32),
            pltpu.SemaphoreType.DMA,
        ],
    )(match_flat, logits_flat)

    s = jnp.sum(partials[:, :L])
    n = jnp.sum(partials[:, L:])
    return jnp.where(n > 0, s / jnp.maximum(n, 1.0), jnp.float32(0.0))


# speedup vs baseline: 39.4066x; 39.4066x over previous
import jax
import jax.numpy as jnp
from jax import lax
from jax.experimental import pallas as pl
from jax.experimental.pallas import tpu as pltpu
from jax.experimental.pallas import tpu_sc as plsc

B = 8
A = 262144
N = B * A
G = 128
NC = 2
NS = 16
NW = NC * NS
PER_W = N // NW
CH = 8192
NCHUNK = PER_W // CH
L = 16

_LOG1P = (
    3.5075520536942406e-06,
    0.999792435728606,
    -0.49697791116761014,
    0.31459053537083104,
    -0.18878267362071732,
    0.08172680837495,
    -0.017208061121084715,
)


def _ce_body(match_hbm, logits_hbm, out_hbm, m_buf, l_buf, stage, sem):
    wid = lax.axis_index("s") * NC + lax.axis_index("c")
    base = wid * PER_W

    sum_vec = jnp.zeros((L,), jnp.float32)
    cnt_vec = jnp.zeros((L,), jnp.float32)

    for c in range(NCHUNK):
        off = base + c * CH
        pltpu.sync_copy(match_hbm.at[pl.ds(off, CH)], m_buf)
        pltpu.sync_copy(logits_hbm.at[pl.ds(2 * off, 2 * CH)], l_buf)

        def body(i, carry):
            s, n = carry
            for j in range(G // L):
                m = m_buf[pl.ds(i * G + j * L, L)]
                l0 = l_buf[pl.ds(i * (2 * G) + j * L, L)]
                l1 = l_buf[pl.ds(i * (2 * G) + G + j * L, L)]
                d = l0 - l1
                ax = jnp.abs(d)
                t = jnp.exp(-ax)
                p = _LOG1P[6]
                for k in (5, 4, 3, 2, 1, 0):
                    p = p * t + _LOG1P[k]
                r = jnp.where((m == 1) == (d > 0), ax, 0.0)
                ce = p + r
                valid = m != 0
                s = s + jnp.where(valid, ce, 0.0)
                n = n + jnp.where(valid, 1.0, 0.0)
            return s, n

        sum_vec, cnt_vec = lax.fori_loop(0, CH // G, body, (sum_vec, cnt_vec))

    stage[pl.ds(0, L)] = sum_vec
    stage[pl.ds(L, L)] = cnt_vec
    pltpu.sync_copy(stage, out_hbm.at[wid])


@jax.jit
def kernel(rpn_match, rpn_class_logits):
    match_flat = rpn_match.reshape(N)
    logits_flat = (
        rpn_class_logits.reshape(B, A // G, G, 2)
        .transpose(0, 1, 3, 2)
        .reshape(2 * N)
    )

    mesh = plsc.VectorSubcoreMesh(core_axis_name="c", subcore_axis_name="s")
    partials = pl.kernel(
        _ce_body,
        out_type=jax.ShapeDtypeStruct((NW, 2 * L), jnp.float32),
        mesh=mesh,
        compiler_params=pltpu.CompilerParams(needs_layout_passes=False),
        scratch_types=[
            pltpu.VMEM((CH,), jnp.int32),
            pltpu.VMEM((2 * CH,), jnp.float32),
            pltpu.VMEM((2 * L,), jnp.float32),
            pltpu.SemaphoreType.DMA,
        ],
    )(match_flat, logits_flat)

    s = jnp.sum(partials[:, :L])
    n = jnp.sum(partials[:, L:])
    return jnp.where(n > 0, s / jnp.maximum(n, 1.0), jnp.float32(0.0))


# speedup vs baseline: 60.4070x; 1.5329x over previous
import jax
import jax.numpy as jnp
from jax import lax
from jax.experimental import pallas as pl
from jax.experimental.pallas import tpu as pltpu
from jax.experimental.pallas import tpu_sc as plsc

B = 8
A = 262144
N = B * A
G = 128
NC = 2
NS = 16
NW = NC * NS
PER_W = N // NW
CH = 8192
NCHUNK = PER_W // CH
L = 16

_LOG1P = (
    0.00014151217537855532,
    0.9954273382579939,
    -0.4640725804471406,
    0.21641043832783918,
    -0.054862852862074235,
)


def _ce_body(match_hbm, logits_hbm, out_hbm,
             m0, m1, lb0, lb1, stage, sm0, sm1, sl0, sl1):
    wid = lax.axis_index("s") * NC + lax.axis_index("c")
    base = wid * PER_W
    m_bufs, l_bufs = (m0, m1), (lb0, lb1)
    m_sems, l_sems = (sm0, sm1), (sl0, sl1)

    def issue(c):
        k = c % 2
        off = base + c * CH
        hm = pltpu.async_copy(match_hbm.at[pl.ds(off, CH)], m_bufs[k], m_sems[k])
        hl = pltpu.async_copy(
            logits_hbm.at[pl.ds(2 * off, 2 * CH)], l_bufs[k], l_sems[k])
        return hm, hl

    sum_vec = jnp.zeros((L,), jnp.float32)
    cnt_vec = jnp.zeros((L,), jnp.float32)

    pending = issue(0)
    for c in range(NCHUNK):
        k = c % 2
        nxt = issue(c + 1) if c + 1 < NCHUNK else None
        pending[0].wait()
        pending[1].wait()
        m_buf, l_buf = m_bufs[k], l_bufs[k]

        def body(i, carry):
            s, n = carry
            for j in range(G // L):
                m = m_buf[pl.ds(i * G + j * L, L)]
                l0 = l_buf[pl.ds(i * (2 * G) + j * L, L)]
                l1 = l_buf[pl.ds(i * (2 * G) + G + j * L, L)]
                d = l0 - l1
                t = jnp.exp(jnp.minimum(d, -d))
                p = _LOG1P[4]
                for q in (3, 2, 1, 0):
                    p = p * t + _LOG1P[q]
                lc = jnp.where(m == 1, l1, l0)
                ce = jnp.maximum(l0, l1) - lc + p
                vm = jnp.where(m != 0, 1.0, 0.0)
                s = s + ce * vm
                n = n + vm
            return s, n

        sum_vec, cnt_vec = lax.fori_loop(0, CH // G, body, (sum_vec, cnt_vec))
        pending = nxt

    stage[pl.ds(0, L)] = sum_vec
    stage[pl.ds(L, L)] = cnt_vec
    pltpu.sync_copy(stage, out_hbm.at[wid])


@jax.jit
def kernel(rpn_match, rpn_class_logits):
    match_flat = rpn_match.reshape(N)
    logits_flat = (
        rpn_class_logits.reshape(B, A // G, G, 2)
        .transpose(0, 1, 3, 2)
        .reshape(2 * N)
    )

    mesh = plsc.VectorSubcoreMesh(core_axis_name="c", subcore_axis_name="s")
    partials = pl.kernel(
        _ce_body,
        out_type=jax.ShapeDtypeStruct((NW, 2 * L), jnp.float32),
        mesh=mesh,
        compiler_params=pltpu.CompilerParams(needs_layout_passes=False),
        scratch_types=[
            pltpu.VMEM((CH,), jnp.int32),
            pltpu.VMEM((CH,), jnp.int32),
            pltpu.VMEM((2 * CH,), jnp.float32),
            pltpu.VMEM((2 * CH,), jnp.float32),
            pltpu.VMEM((2 * L,), jnp.float32),
            pltpu.SemaphoreType.DMA,
            pltpu.SemaphoreType.DMA,
            pltpu.SemaphoreType.DMA,
            pltpu.SemaphoreType.DMA,
        ],
    )(match_flat, logits_flat)

    s = jnp.sum(partials[:, :L])
    n = jnp.sum(partials[:, L:])
    return jnp.where(n > 0, s / jnp.maximum(n, 1.0), jnp.float32(0.0))
